# baseline (speedup 1.0000x reference)
import jax
import jax.numpy as jnp
from jax import lax
from jax.experimental import pallas as pl
from jax.experimental.pallas import tpu as pltpu
from jax.experimental.pallas import tpu_sc as plsc

BATCH = 16384
FACTORS = 32
LANES = 16
NUM_CORES = 2
NUM_SUBCORES = 16
NW = NUM_CORES * NUM_SUBCORES
B_PER_W = BATCH // NW
CHUNK = 128
N_CHUNKS = B_PER_W // CHUNK
GROUPS = B_PER_W // LANES


def _body(users_r, items_r, user_table, item_table, out_hbm,
          uidx_v, iidx_v, urows_v, irows_v, tbuf_v, out_v, sem):
    wid = lax.axis_index("s") * NUM_CORES + lax.axis_index("c")

    pltpu.sync_copy(users_r.at[wid], uidx_v)
    pltpu.sync_copy(items_r.at[wid], iidx_v)

    copies = []
    for c in range(N_CHUNKS):
        dst_u = urows_v.at[pl.ds(c * CHUNK, CHUNK)]
        dst_i = irows_v.at[pl.ds(c * CHUNK, CHUNK)]
        copies.append(pltpu.async_copy(user_table.at[uidx_v.at[c]], dst_u, sem))
        copies.append(pltpu.async_copy(item_table.at[iidx_v.at[c]], dst_i, sem))
    for cp in copies:
        cp.wait()

    lane_iota = lax.iota(jnp.int32, LANES)

    def group(g, _):
        base = g * LANES
        for j in range(LANES):
            u0 = urows_v[base + j, pl.ds(0, LANES)]
            u1 = urows_v[base + j, pl.ds(LANES, LANES)]
            v0 = irows_v[base + j, pl.ds(0, LANES)]
            v1 = irows_v[base + j, pl.ds(LANES, LANES)]
            tbuf_v[pl.ds(j * LANES, LANES)] = u0 * v0 + u1 * v1
        acc = jnp.zeros((LANES,), jnp.float32)
        for l in range(LANES):
            acc = acc + plsc.load_gather(tbuf_v, [lane_iota * LANES + l])
        out_v[pl.ds(base, LANES)] = acc
        return 0

    lax.fori_loop(0, GROUPS, group, 0)

    pltpu.sync_copy(out_v, out_hbm.at[pl.ds(wid * B_PER_W, B_PER_W)])


@jax.jit
def kernel(users_id, items_id, user_table, item_table):
    users_r = users_id.reshape(NW, N_CHUNKS, CHUNK)
    items_r = items_id.reshape(NW, N_CHUNKS, CHUNK)

    mesh = plsc.VectorSubcoreMesh(
        core_axis_name="c", subcore_axis_name="s",
        num_cores=NUM_CORES, num_subcores=NUM_SUBCORES)

    run = pl.kernel(
        _body,
        out_type=jax.ShapeDtypeStruct((BATCH,), jnp.float32),
        mesh=mesh,
        compiler_params=pltpu.CompilerParams(
            needs_layout_passes=False, use_tc_tiling_on_sc=False),
        scratch_types=[
            pltpu.VMEM((N_CHUNKS, CHUNK), jnp.int32),
            pltpu.VMEM((N_CHUNKS, CHUNK), jnp.int32),
            pltpu.VMEM((B_PER_W, FACTORS), jnp.float32),
            pltpu.VMEM((B_PER_W, FACTORS), jnp.float32),
            pltpu.VMEM((LANES * LANES,), jnp.float32),
            pltpu.VMEM((B_PER_W,), jnp.float32),
            pltpu.SemaphoreType.DMA,
        ],
    )
    return run(users_r, items_r, user_table, item_table)
